# Initial kernel scaffold; baseline (speedup 1.0000x reference)
import jax
import jax.numpy as jnp
from jax.experimental import pallas as pl


def kernel(obs, act, rewards, group_ids, logp_old, pW1, pb1, pW2, pb2, vW1, vb1, vW2, vb2, clip_eps, ent_coef, beta_kl, ref_model):
    raise NotImplementedError("write your pallas kernel here")


# speedup vs baseline: 6.7217x; 6.7217x over previous
import functools

import jax
import jax.numpy as jnp
from jax import lax
from jax.experimental import pallas as pl
from jax.experimental.pallas import tpu as pltpu
from jax.experimental.pallas import tpu_sc as plsc

N = 32768
OBS_DIM = 256
ACT_DIM = 64
G = 2048
HID = 64

NS = 16
CH = N // NS
GS = G // NS
L = 16
SCW = 128


def _sqrt_nwt(x):
    y = 0.5 * (x + 1.0)
    for _ in range(18):
        y = 0.5 * (y + x / y)
    return y


def _sc_adv_body(rew_hbm, gidsf_hbm, adv_hbm,
                 r_v, q_v, idsf_v, idsc_v, ones_v,
                 sloc, cloc, qloc, bloc, gloc, tmp16, redloc,
                 bfull, gfull, adv_v,
                 s_sh, c_sh, q_sh, b_sh, g_sh, red_sh):
    w = lax.axis_index("s")
    base = w * CH
    gbase = w * GS

    pltpu.sync_copy(rew_hbm.at[pl.ds(base, CH)], r_v)
    pltpu.sync_copy(gidsf_hbm.at[pl.ds(base, CH)], idsf_v)
    for j in range(CH // SCW):
        pltpu.sync_copy(gidsf_hbm.at[pl.ds(base + j * SCW, SCW)],
                        idsc_v.at[j])

    ones16 = jnp.full((L,), 1.0, jnp.float32)
    zeros16 = jnp.zeros((L,), jnp.float32)
    for kk in range(SCW // L):
        ones_v[pl.ds(L * kk, L)] = ones16

    def sq_body(i, c):
        r = r_v[pl.ds(i * L, L)]
        q_v[pl.ds(i * L, L)] = r * r
        return c

    lax.fori_loop(0, CH // L, sq_body, 0)

    for kk in range(GS // L):
        bloc[pl.ds(L * kk, L)] = zeros16
    pltpu.sync_copy(bloc, s_sh.at[pl.ds(gbase, GS)])
    pltpu.sync_copy(bloc, c_sh.at[pl.ds(gbase, GS)])
    pltpu.sync_copy(bloc, q_sh.at[pl.ds(gbase, GS)])

    @pl.when(w == 0)
    def _():
        tmp16[...] = zeros16
        pltpu.sync_copy(tmp16, red_sh)

    plsc.subcore_barrier()

    def sc_body(j, c):
        idx = idsc_v.at[j]
        pltpu.sync_copy(r_v.at[pl.ds(j * SCW, SCW)], s_sh.at[idx], add=True)
        pltpu.sync_copy(ones_v, c_sh.at[idx], add=True)
        pltpu.sync_copy(q_v.at[pl.ds(j * SCW, SCW)], q_sh.at[idx], add=True)
        return c

    lax.fori_loop(0, CH // SCW, sc_body, 0)
    plsc.subcore_barrier()

    pltpu.sync_copy(s_sh.at[pl.ds(gbase, GS)], sloc)
    pltpu.sync_copy(c_sh.at[pl.ds(gbase, GS)], cloc)
    pltpu.sync_copy(q_sh.at[pl.ds(gbase, GS)], qloc)
    acc1 = zeros16
    acc2 = zeros16
    for kk in range(GS // L):
        dsl = pl.ds(L * kk, L)
        s = sloc[dsl]
        c = cloc[dsl]
        q = qloc[dsl]
        cd = c + 1e-8
        b = s / cd
        m0 = q - 2.0 * b * s + b * b * c
        gi = 1.0 / _sqrt_nwt(m0 / cd + 1e-8)
        bloc[dsl] = b
        gloc[dsl] = gi
        acc1 = acc1 + (s - c * b) * gi
        acc2 = acc2 + m0 * gi * gi
    pltpu.sync_copy(bloc, b_sh.at[pl.ds(gbase, GS)])
    pltpu.sync_copy(gloc, g_sh.at[pl.ds(gbase, GS)])
    idx0 = lax.iota(jnp.int32, L) * 0
    tmp16[...] = acc1
    pltpu.sync_copy(tmp16, red_sh.at[idx0], add=True)
    tmp16[...] = acc2
    pltpu.sync_copy(tmp16, red_sh.at[idx0 + 1], add=True)
    plsc.subcore_barrier()

    pltpu.sync_copy(red_sh, redloc)
    meanvec = plsc.load_gather(redloc, [idx0]) * (1.0 / N)
    sadv2vec = plsc.load_gather(redloc, [idx0 + 1])
    varvec = sadv2vec * (1.0 / N) - meanvec * meanvec
    stdvec = _sqrt_nwt(varvec)
    cvec = meanvec / (stdvec + 1e-8)

    pltpu.sync_copy(b_sh, bfull)
    pltpu.sync_copy(g_sh, gfull)

    def adv_body(i, cv):
        ids = idsf_v[pl.ds(i * L, L)]
        r = r_v[pl.ds(i * L, L)]
        bg = plsc.load_gather(bfull, [ids])
        gg = plsc.load_gather(gfull, [ids])
        adv_v[pl.ds(i * L, L)] = (r - bg) * gg - cv
        return cv

    lax.fori_loop(0, CH // L, adv_body, cvec)
    pltpu.sync_copy(adv_v, adv_hbm.at[pl.ds(base, CH)])


def _sc_adv_call(rewards, gids_flat):
    mesh = plsc.VectorSubcoreMesh(core_axis_name="c", subcore_axis_name="s",
                                  num_cores=1, num_subcores=NS)
    f = pl.kernel(
        _sc_adv_body,
        out_type=jax.ShapeDtypeStruct((N,), jnp.float32),
        mesh=mesh,
        compiler_params=pltpu.CompilerParams(needs_layout_passes=False),
        scratch_types=[
            pltpu.VMEM((CH,), jnp.float32),
            pltpu.VMEM((CH,), jnp.float32),
            pltpu.VMEM((CH,), jnp.int32),
            pltpu.VMEM((CH // SCW, SCW), jnp.int32),
            pltpu.VMEM((SCW,), jnp.float32),
            pltpu.VMEM((GS,), jnp.float32),
            pltpu.VMEM((GS,), jnp.float32),
            pltpu.VMEM((GS,), jnp.float32),
            pltpu.VMEM((GS,), jnp.float32),
            pltpu.VMEM((GS,), jnp.float32),
            pltpu.VMEM((L,), jnp.float32),
            pltpu.VMEM((L,), jnp.float32),
            pltpu.VMEM((G,), jnp.float32),
            pltpu.VMEM((G,), jnp.float32),
            pltpu.VMEM((CH,), jnp.float32),
            pltpu.VMEM_SHARED((G,), jnp.float32),
            pltpu.VMEM_SHARED((G,), jnp.float32),
            pltpu.VMEM_SHARED((G,), jnp.float32),
            pltpu.VMEM_SHARED((G,), jnp.float32),
            pltpu.VMEM_SHARED((G,), jnp.float32),
            pltpu.VMEM_SHARED((L,), jnp.float32),
        ],
    )
    return f(rewards, gids_flat)


BLK = 1024
NB = N // BLK


def _tc_loss_body(obs_ref, act_ref, lpo_ref, adv_ref, w1_ref, b1_ref,
                  w2_ref, b2_ref, lohi_ref, pg_ref, ent_ref):
    i = pl.program_id(0)
    x = obs_ref[...]
    h = jnp.tanh(jnp.dot(x, w1_ref[...], preferred_element_type=jnp.float32)
                 + b1_ref[...])
    logits = (jnp.dot(h, w2_ref[...], preferred_element_type=jnp.float32)
              + b2_ref[...])
    m = jnp.max(logits, axis=-1, keepdims=True)
    e = jnp.exp(logits - m)
    se = jnp.sum(e, axis=-1, keepdims=True)
    lse = jnp.log(se) + m
    oh = lax.broadcasted_iota(jnp.int32, (BLK, ACT_DIM), 1) == act_ref[...]
    picked = jnp.sum(jnp.where(oh, logits, 0.0), axis=-1, keepdims=True)
    logp_new = picked - lse
    ent_blk = jnp.sum(lse - jnp.sum(e * logits, axis=-1, keepdims=True) / se)
    ratio = jnp.exp(logp_new - lpo_ref[...])
    adv = adv_ref[...]
    lo = lohi_ref[0, 0]
    hi = lohi_ref[0, 1]
    s1 = ratio * adv
    s2 = jnp.clip(ratio, lo, hi) * adv
    pg_blk = jnp.sum(jnp.minimum(s1, s2))

    @pl.when(i == 0)
    def _():
        pg_ref[0, 0] = 0.0
        ent_ref[0, 0] = 0.0

    pg_ref[0, 0] += pg_blk
    ent_ref[0, 0] += ent_blk


def _tc_loss_call(obs, act2, lpo2, adv2, pW1, pb1r, pW2, pb2r, lohi):
    return pl.pallas_call(
        _tc_loss_body,
        grid=(NB,),
        in_specs=[
            pl.BlockSpec((BLK, OBS_DIM), lambda i: (i, 0)),
            pl.BlockSpec((BLK, 1), lambda i: (i, 0)),
            pl.BlockSpec((BLK, 1), lambda i: (i, 0)),
            pl.BlockSpec((BLK, 1), lambda i: (i, 0)),
            pl.BlockSpec((OBS_DIM, HID), lambda i: (0, 0)),
            pl.BlockSpec((1, HID), lambda i: (0, 0)),
            pl.BlockSpec((HID, ACT_DIM), lambda i: (0, 0)),
            pl.BlockSpec((1, ACT_DIM), lambda i: (0, 0)),
            pl.BlockSpec((1, 2), lambda i: (0, 0), memory_space=pltpu.SMEM),
        ],
        out_specs=[
            pl.BlockSpec((1, 1), lambda i: (0, 0), memory_space=pltpu.SMEM),
            pl.BlockSpec((1, 1), lambda i: (0, 0), memory_space=pltpu.SMEM),
        ],
        out_shape=[
            jax.ShapeDtypeStruct((1, 1), jnp.float32),
            jax.ShapeDtypeStruct((1, 1), jnp.float32),
        ],
        compiler_params=pltpu.CompilerParams(
            dimension_semantics=("arbitrary",),
        ),
    )(obs, act2, lpo2, adv2, pW1, pb1r, pW2, pb2r, lohi)


def kernel(obs, act, rewards, group_ids, logp_old, pW1, pb1, pW2, pb2,
           vW1, vb1, vW2, vb2, clip_eps, ent_coef, beta_kl, ref_model):
    gids = group_ids.astype(jnp.int32)
    adv = _sc_adv_call(rewards, gids)
    ce = jnp.asarray(clip_eps, jnp.float32)
    lohi = jnp.stack([1.0 - ce, 1.0 + ce]).reshape(1, 2)
    pg_sum, ent_sum = _tc_loss_call(
        obs,
        act.astype(jnp.int32).reshape(N, 1),
        logp_old.reshape(N, 1),
        adv.reshape(N, 1),
        pW1,
        pb1.reshape(1, HID),
        pW2,
        pb2.reshape(1, ACT_DIM),
        lohi,
    )
    ec = jnp.asarray(ent_coef, jnp.float32)
    return -(pg_sum[0, 0] + ec * ent_sum[0, 0]) / N


# speedup vs baseline: 9.1163x; 1.3563x over previous
import functools

import jax
import jax.numpy as jnp
from jax import lax
from jax.experimental import pallas as pl
from jax.experimental.pallas import tpu as pltpu
from jax.experimental.pallas import tpu_sc as plsc

N = 32768
OBS_DIM = 256
ACT_DIM = 64
G = 2048
HID = 64

NS = 16
CH = N // NS
GS = G // NS
L = 16
SCW = 128


def _sqrt_nwt(x):
    y = 0.5 * (x + 1.0)
    for _ in range(18):
        y = 0.5 * (y + x / y)
    return y


def _sc_adv_body(rew_hbm, gidsf_hbm, adv_hbm,
                 r_v, q_v, idsf_v, idsc_v, ones_v,
                 sloc, cloc, qloc, bloc, gloc, tmp16, redloc,
                 bfull, gfull, adv_v,
                 s_sh, c_sh, q_sh, b_sh, g_sh, red_sh):
    w = lax.axis_index("s")
    base = w * CH
    gbase = w * GS

    pltpu.sync_copy(rew_hbm.at[pl.ds(base, CH)], r_v)
    pltpu.sync_copy(gidsf_hbm.at[pl.ds(base, CH)], idsf_v)
    for j in range(CH // SCW):
        pltpu.sync_copy(gidsf_hbm.at[pl.ds(base + j * SCW, SCW)],
                        idsc_v.at[j])

    ones16 = jnp.full((L,), 1.0, jnp.float32)
    zeros16 = jnp.zeros((L,), jnp.float32)
    for kk in range(SCW // L):
        ones_v[pl.ds(L * kk, L)] = ones16

    def sq_body(i, c):
        r = r_v[pl.ds(i * L, L)]
        q_v[pl.ds(i * L, L)] = r * r
        return c

    lax.fori_loop(0, CH // L, sq_body, 0)

    for kk in range(GS // L):
        bloc[pl.ds(L * kk, L)] = zeros16
    pltpu.sync_copy(bloc, s_sh.at[pl.ds(gbase, GS)])
    pltpu.sync_copy(bloc, c_sh.at[pl.ds(gbase, GS)])
    pltpu.sync_copy(bloc, q_sh.at[pl.ds(gbase, GS)])

    @pl.when(w == 0)
    def _():
        tmp16[...] = zeros16
        pltpu.sync_copy(tmp16, red_sh)

    plsc.subcore_barrier()

    def sc_body(j, c):
        idx = idsc_v.at[j]
        pltpu.sync_copy(r_v.at[pl.ds(j * SCW, SCW)], s_sh.at[idx], add=True)
        pltpu.sync_copy(ones_v, c_sh.at[idx], add=True)
        pltpu.sync_copy(q_v.at[pl.ds(j * SCW, SCW)], q_sh.at[idx], add=True)
        return c

    lax.fori_loop(0, CH // SCW, sc_body, 0)
    plsc.subcore_barrier()

    pltpu.sync_copy(s_sh.at[pl.ds(gbase, GS)], sloc)
    pltpu.sync_copy(c_sh.at[pl.ds(gbase, GS)], cloc)
    pltpu.sync_copy(q_sh.at[pl.ds(gbase, GS)], qloc)
    acc1 = zeros16
    acc2 = zeros16
    for kk in range(GS // L):
        dsl = pl.ds(L * kk, L)
        s = sloc[dsl]
        c = cloc[dsl]
        q = qloc[dsl]
        cd = c + 1e-8
        b = s / cd
        m0 = q - 2.0 * b * s + b * b * c
        gi = 1.0 / _sqrt_nwt(m0 / cd + 1e-8)
        bloc[dsl] = b
        gloc[dsl] = gi
        acc1 = acc1 + (s - c * b) * gi
        acc2 = acc2 + m0 * gi * gi
    pltpu.sync_copy(bloc, b_sh.at[pl.ds(gbase, GS)])
    pltpu.sync_copy(gloc, g_sh.at[pl.ds(gbase, GS)])
    idx0 = lax.iota(jnp.int32, L) * 0
    tmp16[...] = acc1
    pltpu.sync_copy(tmp16, red_sh.at[idx0], add=True)
    tmp16[...] = acc2
    pltpu.sync_copy(tmp16, red_sh.at[idx0 + 1], add=True)
    plsc.subcore_barrier()

    pltpu.sync_copy(red_sh, redloc)
    meanvec = plsc.load_gather(redloc, [idx0]) * (1.0 / N)
    sadv2vec = plsc.load_gather(redloc, [idx0 + 1])
    varvec = sadv2vec * (1.0 / N) - meanvec * meanvec
    stdvec = _sqrt_nwt(varvec)
    cvec = meanvec / (stdvec + 1e-8)

    pltpu.sync_copy(b_sh, bfull)
    pltpu.sync_copy(g_sh, gfull)

    def adv_body(i, cv):
        ids = idsf_v[pl.ds(i * L, L)]
        r = r_v[pl.ds(i * L, L)]
        bg = plsc.load_gather(bfull, [ids])
        gg = plsc.load_gather(gfull, [ids])
        adv_v[pl.ds(i * L, L)] = (r - bg) * gg - cv
        return cv

    lax.fori_loop(0, CH // L, adv_body, cvec)
    pltpu.sync_copy(adv_v, adv_hbm.at[pl.ds(base, CH)])


def _sc_adv_call(rewards, gids_flat):
    mesh = plsc.VectorSubcoreMesh(core_axis_name="c", subcore_axis_name="s",
                                  num_cores=1, num_subcores=NS)
    f = pl.kernel(
        _sc_adv_body,
        out_type=jax.ShapeDtypeStruct((N,), jnp.float32),
        mesh=mesh,
        compiler_params=pltpu.CompilerParams(needs_layout_passes=False),
        scratch_types=[
            pltpu.VMEM((CH,), jnp.float32),
            pltpu.VMEM((CH,), jnp.float32),
            pltpu.VMEM((CH,), jnp.int32),
            pltpu.VMEM((CH // SCW, SCW), jnp.int32),
            pltpu.VMEM((SCW,), jnp.float32),
            pltpu.VMEM((GS,), jnp.float32),
            pltpu.VMEM((GS,), jnp.float32),
            pltpu.VMEM((GS,), jnp.float32),
            pltpu.VMEM((GS,), jnp.float32),
            pltpu.VMEM((GS,), jnp.float32),
            pltpu.VMEM((L,), jnp.float32),
            pltpu.VMEM((L,), jnp.float32),
            pltpu.VMEM((G,), jnp.float32),
            pltpu.VMEM((G,), jnp.float32),
            pltpu.VMEM((CH,), jnp.float32),
            pltpu.VMEM_SHARED((G,), jnp.float32),
            pltpu.VMEM_SHARED((G,), jnp.float32),
            pltpu.VMEM_SHARED((G,), jnp.float32),
            pltpu.VMEM_SHARED((G,), jnp.float32),
            pltpu.VMEM_SHARED((G,), jnp.float32),
            pltpu.VMEM_SHARED((L,), jnp.float32),
        ],
    )
    return f(rewards, gids_flat)


BLK = 1024
NB = N // BLK


def _tc_loss_body(obs_ref, act_ref, lpo_ref, adv_ref, w1_ref, b1_ref,
                  w2_ref, b2c_ref, lohi_ref, pg_ref, ent_ref):
    i = pl.program_id(0)
    x = obs_ref[...]
    h = jnp.tanh(jnp.dot(x, w1_ref[...], preferred_element_type=jnp.float32)
                 + b1_ref[...])
    lt = lax.dot_general(w2_ref[...], h, (((0,), (1,)), ((), ())),
                         preferred_element_type=jnp.float32) + b2c_ref[...]
    m = jnp.max(lt, axis=0, keepdims=True)
    e = jnp.exp(lt - m)
    se = jnp.sum(e, axis=0, keepdims=True)
    lse = jnp.log(se) + m
    act_row = act_ref[...].reshape(1, BLK)
    oh = lax.broadcasted_iota(jnp.int32, (ACT_DIM, BLK), 0) == act_row
    picked = jnp.sum(jnp.where(oh, lt, 0.0), axis=0, keepdims=True)
    ent_blk = jnp.sum(lse - jnp.sum(e * lt, axis=0, keepdims=True) / se)
    ratio = jnp.exp(picked - lse - lpo_ref[...].reshape(1, BLK))
    adv = adv_ref[...].reshape(1, BLK)
    lo = lohi_ref[0, 0]
    hi = lohi_ref[0, 1]
    s1 = ratio * adv
    s2 = jnp.clip(ratio, lo, hi) * adv
    pg_blk = jnp.sum(jnp.minimum(s1, s2))

    @pl.when(i == 0)
    def _():
        pg_ref[0, 0] = 0.0
        ent_ref[0, 0] = 0.0

    pg_ref[0, 0] += pg_blk
    ent_ref[0, 0] += ent_blk


def _tc_loss_call(obs, act1, lpo1, adv1, pW1, pb1r, pW2, pb2c, lohi):
    return pl.pallas_call(
        _tc_loss_body,
        grid=(NB,),
        in_specs=[
            pl.BlockSpec((BLK, OBS_DIM), lambda i: (i, 0)),
            pl.BlockSpec((BLK,), lambda i: (i,)),
            pl.BlockSpec((BLK,), lambda i: (i,)),
            pl.BlockSpec((BLK,), lambda i: (i,)),
            pl.BlockSpec((OBS_DIM, HID), lambda i: (0, 0)),
            pl.BlockSpec((1, HID), lambda i: (0, 0)),
            pl.BlockSpec((HID, ACT_DIM), lambda i: (0, 0)),
            pl.BlockSpec((ACT_DIM, 1), lambda i: (0, 0)),
            pl.BlockSpec((1, 2), lambda i: (0, 0), memory_space=pltpu.SMEM),
        ],
        out_specs=[
            pl.BlockSpec((1, 1), lambda i: (0, 0), memory_space=pltpu.SMEM),
            pl.BlockSpec((1, 1), lambda i: (0, 0), memory_space=pltpu.SMEM),
        ],
        out_shape=[
            jax.ShapeDtypeStruct((1, 1), jnp.float32),
            jax.ShapeDtypeStruct((1, 1), jnp.float32),
        ],
        compiler_params=pltpu.CompilerParams(
            dimension_semantics=("arbitrary",),
        ),
    )(obs, act1, lpo1, adv1, pW1, pb1r, pW2, pb2c, lohi)


def kernel(obs, act, rewards, group_ids, logp_old, pW1, pb1, pW2, pb2,
           vW1, vb1, vW2, vb2, clip_eps, ent_coef, beta_kl, ref_model):
    gids = group_ids.astype(jnp.int32)
    adv = _sc_adv_call(rewards, gids)
    ce = jnp.asarray(clip_eps, jnp.float32)
    lohi = jnp.stack([1.0 - ce, 1.0 + ce]).reshape(1, 2)
    pg_sum, ent_sum = _tc_loss_call(
        obs,
        act.astype(jnp.int32),
        logp_old,
        adv,
        pW1,
        pb1.reshape(1, HID),
        pW2,
        pb2.reshape(ACT_DIM, 1),
        lohi,
    )
    ec = jnp.asarray(ent_coef, jnp.float32)
    return -(pg_sum[0, 0] + ec * ent_sum[0, 0]) / N


# speedup vs baseline: 10.3860x; 1.1393x over previous
import functools

import jax
import jax.numpy as jnp
from jax import lax
from jax.experimental import pallas as pl
from jax.experimental.pallas import tpu as pltpu
from jax.experimental.pallas import tpu_sc as plsc

N = 32768
OBS_DIM = 256
ACT_DIM = 64
G = 2048
HID = 64

NS = 16
CH = N // NS
GS = G // NS
L = 16
SCW = 128


def _sqrt_nwt(x):
    y = 0.5 * (x + 1.0)
    for _ in range(18):
        y = 0.5 * (y + x / y)
    return y


def _sc_adv_body(rew_hbm, gidsf_hbm, adv_hbm,
                 r_v, q_v, idsf_v, idsc_v, ones_v,
                 sloc, cloc, qloc, bloc, gloc, tmp16, redloc,
                 bfull, gfull, adv_v, sem,
                 s_sh, c_sh, q_sh, b_sh, g_sh, red_sh):
    w = lax.axis_index("s")
    base = w * CH
    gbase = w * GS

    descs = [
        pltpu.async_copy(rew_hbm.at[pl.ds(base, CH)], r_v, sem),
        pltpu.async_copy(gidsf_hbm.at[pl.ds(base, CH)], idsf_v, sem),
    ]
    for j in range(CH // SCW):
        descs.append(pltpu.async_copy(
            gidsf_hbm.at[pl.ds(base + j * SCW, SCW)], idsc_v.at[j], sem))

    ones16 = jnp.full((L,), 1.0, jnp.float32)
    zeros16 = jnp.zeros((L,), jnp.float32)
    for kk in range(SCW // L):
        ones_v[pl.ds(L * kk, L)] = ones16
    for kk in range(GS // L):
        bloc[pl.ds(L * kk, L)] = zeros16
    for d in descs:
        d.wait()

    def sq_body(i, c):
        r = r_v[pl.ds(i * L, L)]
        q_v[pl.ds(i * L, L)] = r * r
        return c

    lax.fori_loop(0, CH // L, sq_body, 0)

    descs = [
        pltpu.async_copy(bloc, s_sh.at[pl.ds(gbase, GS)], sem),
        pltpu.async_copy(bloc, c_sh.at[pl.ds(gbase, GS)], sem),
        pltpu.async_copy(bloc, q_sh.at[pl.ds(gbase, GS)], sem),
    ]

    @pl.when(w == 0)
    def _():
        tmp16[...] = zeros16
        pltpu.sync_copy(tmp16, red_sh)

    for d in descs:
        d.wait()
    plsc.subcore_barrier()

    descs = []
    for j in range(CH // SCW):
        idx = idsc_v.at[j]
        descs.append(pltpu.async_copy(
            r_v.at[pl.ds(j * SCW, SCW)], s_sh.at[idx], sem, add=True))
        descs.append(pltpu.async_copy(ones_v, c_sh.at[idx], sem, add=True))
        descs.append(pltpu.async_copy(
            q_v.at[pl.ds(j * SCW, SCW)], q_sh.at[idx], sem, add=True))
    for d in descs:
        d.wait()
    plsc.subcore_barrier()

    descs = [
        pltpu.async_copy(s_sh.at[pl.ds(gbase, GS)], sloc, sem),
        pltpu.async_copy(c_sh.at[pl.ds(gbase, GS)], cloc, sem),
        pltpu.async_copy(q_sh.at[pl.ds(gbase, GS)], qloc, sem),
    ]
    for d in descs:
        d.wait()
    acc1 = zeros16
    acc2 = zeros16
    for kk in range(GS // L):
        dsl = pl.ds(L * kk, L)
        s = sloc[dsl]
        c = cloc[dsl]
        q = qloc[dsl]
        cd = c + 1e-8
        b = s / cd
        m0 = q - 2.0 * b * s + b * b * c
        gi = 1.0 / _sqrt_nwt(m0 / cd + 1e-8)
        bloc[dsl] = b
        gloc[dsl] = gi
        acc1 = acc1 + (s - c * b) * gi
        acc2 = acc2 + m0 * gi * gi
    idx0 = lax.iota(jnp.int32, L) * 0
    tmp16[...] = acc1
    redloc[...] = acc2
    descs = [
        pltpu.async_copy(bloc, b_sh.at[pl.ds(gbase, GS)], sem),
        pltpu.async_copy(gloc, g_sh.at[pl.ds(gbase, GS)], sem),
        pltpu.async_copy(tmp16, red_sh.at[idx0], sem, add=True),
        pltpu.async_copy(redloc, red_sh.at[idx0 + 1], sem, add=True),
    ]
    for d in descs:
        d.wait()
    plsc.subcore_barrier()

    pltpu.sync_copy(red_sh, redloc)
    meanvec = plsc.load_gather(redloc, [idx0]) * (1.0 / N)
    sadv2vec = plsc.load_gather(redloc, [idx0 + 1])
    varvec = sadv2vec * (1.0 / N) - meanvec * meanvec
    stdvec = _sqrt_nwt(varvec)
    cvec = meanvec / (stdvec + 1e-8)

    descs = [
        pltpu.async_copy(b_sh, bfull, sem),
        pltpu.async_copy(g_sh, gfull, sem),
    ]
    for d in descs:
        d.wait()

    def adv_body(i, cv):
        ids = idsf_v[pl.ds(i * L, L)]
        r = r_v[pl.ds(i * L, L)]
        bg = plsc.load_gather(bfull, [ids])
        gg = plsc.load_gather(gfull, [ids])
        adv_v[pl.ds(i * L, L)] = (r - bg) * gg - cv
        return cv

    lax.fori_loop(0, CH // L, adv_body, cvec)
    pltpu.sync_copy(adv_v, adv_hbm.at[pl.ds(base, CH)])


def _sc_adv_call(rewards, gids_flat):
    mesh = plsc.VectorSubcoreMesh(core_axis_name="c", subcore_axis_name="s",
                                  num_cores=1, num_subcores=NS)
    f = pl.kernel(
        _sc_adv_body,
        out_type=jax.ShapeDtypeStruct((N,), jnp.float32),
        mesh=mesh,
        compiler_params=pltpu.CompilerParams(needs_layout_passes=False),
        scratch_types=[
            pltpu.VMEM((CH,), jnp.float32),
            pltpu.VMEM((CH,), jnp.float32),
            pltpu.VMEM((CH,), jnp.int32),
            pltpu.VMEM((CH // SCW, SCW), jnp.int32),
            pltpu.VMEM((SCW,), jnp.float32),
            pltpu.VMEM((GS,), jnp.float32),
            pltpu.VMEM((GS,), jnp.float32),
            pltpu.VMEM((GS,), jnp.float32),
            pltpu.VMEM((GS,), jnp.float32),
            pltpu.VMEM((GS,), jnp.float32),
            pltpu.VMEM((L,), jnp.float32),
            pltpu.VMEM((L,), jnp.float32),
            pltpu.VMEM((G,), jnp.float32),
            pltpu.VMEM((G,), jnp.float32),
            pltpu.VMEM((CH,), jnp.float32),
            pltpu.SemaphoreType.DMA,
            pltpu.VMEM_SHARED((G,), jnp.float32),
            pltpu.VMEM_SHARED((G,), jnp.float32),
            pltpu.VMEM_SHARED((G,), jnp.float32),
            pltpu.VMEM_SHARED((G,), jnp.float32),
            pltpu.VMEM_SHARED((G,), jnp.float32),
            pltpu.VMEM_SHARED((L,), jnp.float32),
        ],
    )
    return f(rewards, gids_flat)


BLK = 1024
NB = N // BLK


def _tc_loss_body(obs_ref, act_ref, lpo_ref, adv_ref, w1_ref, b1_ref,
                  w2_ref, b2c_ref, lohi_ref, pg_ref, ent_ref):
    i = pl.program_id(0)
    x = obs_ref[...]
    h = jnp.tanh(jnp.dot(x, w1_ref[...], preferred_element_type=jnp.float32)
                 + b1_ref[...])
    lt = lax.dot_general(w2_ref[...], h, (((0,), (1,)), ((), ())),
                         preferred_element_type=jnp.float32) + b2c_ref[...]
    m = jnp.max(lt, axis=0, keepdims=True)
    e = jnp.exp(lt - m)
    se = jnp.sum(e, axis=0, keepdims=True)
    lse = jnp.log(se) + m
    act_row = act_ref[...].reshape(1, BLK)
    oh = lax.broadcasted_iota(jnp.int32, (ACT_DIM, BLK), 0) == act_row
    picked = jnp.sum(jnp.where(oh, lt, 0.0), axis=0, keepdims=True)
    ent_blk = jnp.sum(lse - jnp.sum(e * lt, axis=0, keepdims=True) / se)
    ratio = jnp.exp(picked - lse - lpo_ref[...].reshape(1, BLK))
    adv = adv_ref[...].reshape(1, BLK)
    lo = lohi_ref[0, 0]
    hi = lohi_ref[0, 1]
    s1 = ratio * adv
    s2 = jnp.clip(ratio, lo, hi) * adv
    pg_blk = jnp.sum(jnp.minimum(s1, s2))

    @pl.when(i == 0)
    def _():
        pg_ref[0, 0] = 0.0
        ent_ref[0, 0] = 0.0

    pg_ref[0, 0] += pg_blk
    ent_ref[0, 0] += ent_blk


def _tc_loss_call(obs, act1, lpo1, adv1, pW1, pb1r, pW2, pb2c, lohi):
    return pl.pallas_call(
        _tc_loss_body,
        grid=(NB,),
        in_specs=[
            pl.BlockSpec((BLK, OBS_DIM), lambda i: (i, 0)),
            pl.BlockSpec((BLK,), lambda i: (i,)),
            pl.BlockSpec((BLK,), lambda i: (i,)),
            pl.BlockSpec((BLK,), lambda i: (i,)),
            pl.BlockSpec((OBS_DIM, HID), lambda i: (0, 0)),
            pl.BlockSpec((1, HID), lambda i: (0, 0)),
            pl.BlockSpec((HID, ACT_DIM), lambda i: (0, 0)),
            pl.BlockSpec((ACT_DIM, 1), lambda i: (0, 0)),
            pl.BlockSpec((1, 2), lambda i: (0, 0), memory_space=pltpu.SMEM),
        ],
        out_specs=[
            pl.BlockSpec((1, 1), lambda i: (0, 0), memory_space=pltpu.SMEM),
            pl.BlockSpec((1, 1), lambda i: (0, 0), memory_space=pltpu.SMEM),
        ],
        out_shape=[
            jax.ShapeDtypeStruct((1, 1), jnp.float32),
            jax.ShapeDtypeStruct((1, 1), jnp.float32),
        ],
        compiler_params=pltpu.CompilerParams(
            dimension_semantics=("arbitrary",),
        ),
    )(obs, act1, lpo1, adv1, pW1, pb1r, pW2, pb2c, lohi)


def kernel(obs, act, rewards, group_ids, logp_old, pW1, pb1, pW2, pb2,
           vW1, vb1, vW2, vb2, clip_eps, ent_coef, beta_kl, ref_model):
    gids = group_ids.astype(jnp.int32)
    adv = _sc_adv_call(rewards, gids)
    ce = jnp.asarray(clip_eps, jnp.float32)
    lohi = jnp.stack([1.0 - ce, 1.0 + ce]).reshape(1, 2)
    pg_sum, ent_sum = _tc_loss_call(
        obs,
        act.astype(jnp.int32),
        logp_old,
        adv,
        pW1,
        pb1.reshape(1, HID),
        pW2,
        pb2.reshape(ACT_DIM, 1),
        lohi,
    )
    ec = jnp.asarray(ent_coef, jnp.float32)
    return -(pg_sum[0, 0] + ec * ent_sum[0, 0]) / N


# speedup vs baseline: 12.4543x; 1.1991x over previous
import functools

import jax
import jax.numpy as jnp
from jax import lax
from jax.experimental import pallas as pl
from jax.experimental.pallas import tpu as pltpu
from jax.experimental.pallas import tpu_sc as plsc

N = 32768
OBS_DIM = 256
ACT_DIM = 64
G = 2048
HID = 64

NS = 16
CH = N // NS
GS = G // NS
L = 16
SCW = 128


def _sqrt_nwt(x):
    y = 0.5 * (x + 1.0)
    for _ in range(18):
        y = 0.5 * (y + x / y)
    return y


def _sc_adv_body(rew_hbm, gidsf_hbm, adv_hbm,
                 r_v, q_v, idsf_v, idsc_v, ones_v,
                 sloc, cloc, qloc, bloc, gloc, tmp16, redloc,
                 bfull, gfull, adv_v, sem,
                 s_sh, c_sh, q_sh, b_sh, g_sh, red_sh):
    w = lax.axis_index("s")
    base = w * CH
    gbase = w * GS

    descs = [
        pltpu.async_copy(rew_hbm.at[pl.ds(base, CH)], r_v, sem),
        pltpu.async_copy(gidsf_hbm.at[pl.ds(base, CH)], idsf_v, sem),
    ]
    for j in range(CH // SCW):
        descs.append(pltpu.async_copy(
            gidsf_hbm.at[pl.ds(base + j * SCW, SCW)], idsc_v.at[j], sem))

    ones16 = jnp.full((L,), 1.0, jnp.float32)
    zeros16 = jnp.zeros((L,), jnp.float32)
    for kk in range(SCW // L):
        ones_v[pl.ds(L * kk, L)] = ones16
    for kk in range(GS // L):
        bloc[pl.ds(L * kk, L)] = zeros16
    for d in descs:
        d.wait()

    def sq_body(i, c):
        r = r_v[pl.ds(i * L, L)]
        q_v[pl.ds(i * L, L)] = r * r
        return c

    lax.fori_loop(0, CH // L, sq_body, 0)

    descs = [
        pltpu.async_copy(bloc, s_sh.at[pl.ds(gbase, GS)], sem),
        pltpu.async_copy(bloc, c_sh.at[pl.ds(gbase, GS)], sem),
        pltpu.async_copy(bloc, q_sh.at[pl.ds(gbase, GS)], sem),
    ]

    @pl.when(w == 0)
    def _():
        tmp16[...] = zeros16
        pltpu.sync_copy(tmp16, red_sh)

    for d in descs:
        d.wait()
    plsc.subcore_barrier()

    descs = []
    for j in range(CH // SCW):
        idx = idsc_v.at[j]
        descs.append(pltpu.async_copy(
            r_v.at[pl.ds(j * SCW, SCW)], s_sh.at[idx], sem, add=True))
        descs.append(pltpu.async_copy(ones_v, c_sh.at[idx], sem, add=True))
        descs.append(pltpu.async_copy(
            q_v.at[pl.ds(j * SCW, SCW)], q_sh.at[idx], sem, add=True))
    for d in descs:
        d.wait()
    plsc.subcore_barrier()

    descs = [
        pltpu.async_copy(s_sh.at[pl.ds(gbase, GS)], sloc, sem),
        pltpu.async_copy(c_sh.at[pl.ds(gbase, GS)], cloc, sem),
        pltpu.async_copy(q_sh.at[pl.ds(gbase, GS)], qloc, sem),
    ]
    for d in descs:
        d.wait()
    acc1 = zeros16
    acc2 = zeros16
    for kk in range(GS // L):
        dsl = pl.ds(L * kk, L)
        s = sloc[dsl]
        c = cloc[dsl]
        q = qloc[dsl]
        cd = c + 1e-8
        b = s / cd
        m0 = q - 2.0 * b * s + b * b * c
        gi = 1.0 / _sqrt_nwt(m0 / cd + 1e-8)
        bloc[dsl] = b
        gloc[dsl] = gi
        acc1 = acc1 + (s - c * b) * gi
        acc2 = acc2 + m0 * gi * gi
    idx0 = lax.iota(jnp.int32, L) * 0
    tmp16[...] = acc1
    redloc[...] = acc2
    descs = [
        pltpu.async_copy(bloc, b_sh.at[pl.ds(gbase, GS)], sem),
        pltpu.async_copy(gloc, g_sh.at[pl.ds(gbase, GS)], sem),
        pltpu.async_copy(tmp16, red_sh.at[idx0], sem, add=True),
        pltpu.async_copy(redloc, red_sh.at[idx0 + 1], sem, add=True),
    ]
    for d in descs:
        d.wait()
    plsc.subcore_barrier()

    pltpu.sync_copy(red_sh, redloc)
    meanvec = plsc.load_gather(redloc, [idx0]) * (1.0 / N)
    sadv2vec = plsc.load_gather(redloc, [idx0 + 1])
    varvec = sadv2vec * (1.0 / N) - meanvec * meanvec
    stdvec = _sqrt_nwt(varvec)
    cvec = meanvec / (stdvec + 1e-8)

    descs = [
        pltpu.async_copy(b_sh, bfull, sem),
        pltpu.async_copy(g_sh, gfull, sem),
    ]
    for d in descs:
        d.wait()

    def adv_body(i, cv):
        ids = idsf_v[pl.ds(i * L, L)]
        r = r_v[pl.ds(i * L, L)]
        bg = plsc.load_gather(bfull, [ids])
        gg = plsc.load_gather(gfull, [ids])
        adv_v[pl.ds(i * L, L)] = (r - bg) * gg - cv
        return cv

    lax.fori_loop(0, CH // L, adv_body, cvec)
    pltpu.sync_copy(adv_v, adv_hbm.at[pl.ds(base, CH)])


def _sc_adv_call(rewards, gids_flat):
    mesh = plsc.VectorSubcoreMesh(core_axis_name="c", subcore_axis_name="s",
                                  num_cores=1, num_subcores=NS)
    f = pl.kernel(
        _sc_adv_body,
        out_type=jax.ShapeDtypeStruct((N,), jnp.float32),
        mesh=mesh,
        compiler_params=pltpu.CompilerParams(needs_layout_passes=False),
        scratch_types=[
            pltpu.VMEM((CH,), jnp.float32),
            pltpu.VMEM((CH,), jnp.float32),
            pltpu.VMEM((CH,), jnp.int32),
            pltpu.VMEM((CH // SCW, SCW), jnp.int32),
            pltpu.VMEM((SCW,), jnp.float32),
            pltpu.VMEM((GS,), jnp.float32),
            pltpu.VMEM((GS,), jnp.float32),
            pltpu.VMEM((GS,), jnp.float32),
            pltpu.VMEM((GS,), jnp.float32),
            pltpu.VMEM((GS,), jnp.float32),
            pltpu.VMEM((L,), jnp.float32),
            pltpu.VMEM((L,), jnp.float32),
            pltpu.VMEM((G,), jnp.float32),
            pltpu.VMEM((G,), jnp.float32),
            pltpu.VMEM((CH,), jnp.float32),
            pltpu.SemaphoreType.DMA,
            pltpu.VMEM_SHARED((G,), jnp.float32),
            pltpu.VMEM_SHARED((G,), jnp.float32),
            pltpu.VMEM_SHARED((G,), jnp.float32),
            pltpu.VMEM_SHARED((G,), jnp.float32),
            pltpu.VMEM_SHARED((G,), jnp.float32),
            pltpu.VMEM_SHARED((L,), jnp.float32),
        ],
    )
    return f(rewards, gids_flat)


BLK = 2048
NB = N // BLK


def _tc_loss_body(obs_ref, act_ref, lpo_ref, adv_ref, w1_ref, b1_ref,
                  w2_ref, b2c_ref, lohi_ref, pg_ref, ent_ref):
    i = pl.program_id(0)
    x = obs_ref[...]
    h = jnp.tanh(jnp.dot(x, w1_ref[...], preferred_element_type=jnp.float32)
                 + b1_ref[...])
    lt = lax.dot_general(w2_ref[...], h, (((0,), (1,)), ((), ())),
                         preferred_element_type=jnp.float32) + b2c_ref[...]
    m = jnp.max(lt, axis=0, keepdims=True)
    e = jnp.exp(lt - m)
    se = jnp.sum(e, axis=0, keepdims=True)
    lse = jnp.log(se) + m
    act_row = act_ref[...].reshape(1, BLK)
    oh = lax.broadcasted_iota(jnp.int32, (ACT_DIM, BLK), 0) == act_row
    picked = jnp.sum(jnp.where(oh, lt, 0.0), axis=0, keepdims=True)
    ent_blk = jnp.sum(lse - jnp.sum(e * lt, axis=0, keepdims=True) / se)
    ratio = jnp.exp(picked - lse - lpo_ref[...].reshape(1, BLK))
    adv = adv_ref[...].reshape(1, BLK)
    lo = lohi_ref[0, 0]
    hi = lohi_ref[0, 1]
    s1 = ratio * adv
    s2 = jnp.clip(ratio, lo, hi) * adv
    pg_blk = jnp.sum(jnp.minimum(s1, s2))

    @pl.when(i == 0)
    def _():
        pg_ref[0, 0] = 0.0
        ent_ref[0, 0] = 0.0

    pg_ref[0, 0] += pg_blk
    ent_ref[0, 0] += ent_blk


def _tc_loss_call(obs, act1, lpo1, adv1, pW1, pb1r, pW2, pb2c, lohi):
    return pl.pallas_call(
        _tc_loss_body,
        grid=(NB,),
        in_specs=[
            pl.BlockSpec((BLK, OBS_DIM), lambda i: (i, 0)),
            pl.BlockSpec((BLK,), lambda i: (i,)),
            pl.BlockSpec((BLK,), lambda i: (i,)),
            pl.BlockSpec((BLK,), lambda i: (i,)),
            pl.BlockSpec((OBS_DIM, HID), lambda i: (0, 0)),
            pl.BlockSpec((1, HID), lambda i: (0, 0)),
            pl.BlockSpec((HID, ACT_DIM), lambda i: (0, 0)),
            pl.BlockSpec((ACT_DIM, 1), lambda i: (0, 0)),
            pl.BlockSpec((1, 2), lambda i: (0, 0), memory_space=pltpu.SMEM),
        ],
        out_specs=[
            pl.BlockSpec((1, 1), lambda i: (0, 0), memory_space=pltpu.SMEM),
            pl.BlockSpec((1, 1), lambda i: (0, 0), memory_space=pltpu.SMEM),
        ],
        out_shape=[
            jax.ShapeDtypeStruct((1, 1), jnp.float32),
            jax.ShapeDtypeStruct((1, 1), jnp.float32),
        ],
        compiler_params=pltpu.CompilerParams(
            dimension_semantics=("arbitrary",),
        ),
    )(obs, act1, lpo1, adv1, pW1, pb1r, pW2, pb2c, lohi)


def kernel(obs, act, rewards, group_ids, logp_old, pW1, pb1, pW2, pb2,
           vW1, vb1, vW2, vb2, clip_eps, ent_coef, beta_kl, ref_model):
    gids = group_ids.astype(jnp.int32)
    adv = _sc_adv_call(rewards, gids)
    ce = jnp.asarray(clip_eps, jnp.float32)
    lohi = jnp.stack([1.0 - ce, 1.0 + ce]).reshape(1, 2)
    pg_sum, ent_sum = _tc_loss_call(
        obs,
        act.astype(jnp.int32),
        logp_old,
        adv,
        pW1,
        pb1.reshape(1, HID),
        pW2,
        pb2.reshape(ACT_DIM, 1),
        lohi,
    )
    ec = jnp.asarray(ent_coef, jnp.float32)
    return -(pg_sum[0, 0] + ec * ent_sum[0, 0]) / N


# speedup vs baseline: 13.5588x; 1.0887x over previous
import functools

import jax
import jax.numpy as jnp
from jax import lax
from jax.experimental import pallas as pl
from jax.experimental.pallas import tpu as pltpu
from jax.experimental.pallas import tpu_sc as plsc

N = 32768
OBS_DIM = 256
ACT_DIM = 64
G = 2048
HID = 64

NS = 16
CH = N // NS
GS = G // NS
L = 16
SCW = 128


def _sqrt_nwt(x):
    y = 0.5 * (x + 1.0)
    for _ in range(18):
        y = 0.5 * (y + x / y)
    return y


def _sc_adv_body(rew_hbm, gidsf_hbm, adv_hbm,
                 r_v, q_v, idsf_v, idsc_v, ones_v,
                 sloc, cloc, qloc, bloc, gloc, tmp16, redloc,
                 bfull, gfull, adv_v, sem,
                 s_sh, c_sh, q_sh, b_sh, g_sh, red_sh):
    w = lax.axis_index("s")
    base = w * CH
    gbase = w * GS

    descs = [
        pltpu.async_copy(rew_hbm.at[pl.ds(base, CH)], r_v, sem),
        pltpu.async_copy(gidsf_hbm.at[pl.ds(base, CH)], idsf_v, sem),
    ]
    for j in range(CH // SCW):
        descs.append(pltpu.async_copy(
            gidsf_hbm.at[pl.ds(base + j * SCW, SCW)], idsc_v.at[j], sem))

    ones16 = jnp.full((L,), 1.0, jnp.float32)
    zeros16 = jnp.zeros((L,), jnp.float32)
    for kk in range(SCW // L):
        ones_v[pl.ds(L * kk, L)] = ones16
    for kk in range(GS // L):
        bloc[pl.ds(L * kk, L)] = zeros16
    for d in descs:
        d.wait()

    def sq_body(i, c):
        r = r_v[pl.ds(i * L, L)]
        q_v[pl.ds(i * L, L)] = r * r
        return c

    lax.fori_loop(0, CH // L, sq_body, 0)

    descs = [
        pltpu.async_copy(bloc, s_sh.at[pl.ds(gbase, GS)], sem),
        pltpu.async_copy(bloc, c_sh.at[pl.ds(gbase, GS)], sem),
        pltpu.async_copy(bloc, q_sh.at[pl.ds(gbase, GS)], sem),
    ]

    @pl.when(w == 0)
    def _():
        tmp16[...] = zeros16
        pltpu.sync_copy(tmp16, red_sh)

    for d in descs:
        d.wait()
    plsc.subcore_barrier()

    descs = []
    for j in range(CH // SCW):
        idx = idsc_v.at[j]
        descs.append(pltpu.async_copy(
            r_v.at[pl.ds(j * SCW, SCW)], s_sh.at[idx], sem, add=True))
        descs.append(pltpu.async_copy(ones_v, c_sh.at[idx], sem, add=True))
        descs.append(pltpu.async_copy(
            q_v.at[pl.ds(j * SCW, SCW)], q_sh.at[idx], sem, add=True))
    for d in descs:
        d.wait()
    plsc.subcore_barrier()

    descs = [
        pltpu.async_copy(s_sh.at[pl.ds(gbase, GS)], sloc, sem),
        pltpu.async_copy(c_sh.at[pl.ds(gbase, GS)], cloc, sem),
        pltpu.async_copy(q_sh.at[pl.ds(gbase, GS)], qloc, sem),
    ]
    for d in descs:
        d.wait()
    acc1 = zeros16
    acc2 = zeros16
    for kk in range(GS // L):
        dsl = pl.ds(L * kk, L)
        s = sloc[dsl]
        c = cloc[dsl]
        q = qloc[dsl]
        cd = c + 1e-8
        b = s / cd
        m0 = q - 2.0 * b * s + b * b * c
        gi = 1.0 / _sqrt_nwt(m0 / cd + 1e-8)
        bloc[dsl] = b
        gloc[dsl] = gi
        acc1 = acc1 + (s - c * b) * gi
        acc2 = acc2 + m0 * gi * gi
    idx0 = lax.iota(jnp.int32, L) * 0
    tmp16[...] = acc1
    redloc[...] = acc2
    descs = [
        pltpu.async_copy(bloc, b_sh.at[pl.ds(gbase, GS)], sem),
        pltpu.async_copy(gloc, g_sh.at[pl.ds(gbase, GS)], sem),
        pltpu.async_copy(tmp16, red_sh.at[idx0], sem, add=True),
        pltpu.async_copy(redloc, red_sh.at[idx0 + 1], sem, add=True),
    ]
    for d in descs:
        d.wait()
    plsc.subcore_barrier()

    pltpu.sync_copy(red_sh, redloc)
    meanvec = plsc.load_gather(redloc, [idx0]) * (1.0 / N)
    sadv2vec = plsc.load_gather(redloc, [idx0 + 1])
    varvec = sadv2vec * (1.0 / N) - meanvec * meanvec
    stdvec = _sqrt_nwt(varvec)
    cvec = meanvec / (stdvec + 1e-8)

    descs = [
        pltpu.async_copy(b_sh, bfull, sem),
        pltpu.async_copy(g_sh, gfull, sem),
    ]
    for d in descs:
        d.wait()

    def adv_body(i, cv):
        ids = idsf_v[pl.ds(i * L, L)]
        r = r_v[pl.ds(i * L, L)]
        bg = plsc.load_gather(bfull, [ids])
        gg = plsc.load_gather(gfull, [ids])
        adv_v[pl.ds(i * L, L)] = (r - bg) * gg - cv
        return cv

    lax.fori_loop(0, CH // L, adv_body, cvec)
    pltpu.sync_copy(adv_v, adv_hbm.at[pl.ds(base, CH)])


def _sc_adv_call(rewards, gids_flat):
    mesh = plsc.VectorSubcoreMesh(core_axis_name="c", subcore_axis_name="s",
                                  num_cores=1, num_subcores=NS)
    f = pl.kernel(
        _sc_adv_body,
        out_type=jax.ShapeDtypeStruct((N,), jnp.float32),
        mesh=mesh,
        compiler_params=pltpu.CompilerParams(needs_layout_passes=False),
        scratch_types=[
            pltpu.VMEM((CH,), jnp.float32),
            pltpu.VMEM((CH,), jnp.float32),
            pltpu.VMEM((CH,), jnp.int32),
            pltpu.VMEM((CH // SCW, SCW), jnp.int32),
            pltpu.VMEM((SCW,), jnp.float32),
            pltpu.VMEM((GS,), jnp.float32),
            pltpu.VMEM((GS,), jnp.float32),
            pltpu.VMEM((GS,), jnp.float32),
            pltpu.VMEM((GS,), jnp.float32),
            pltpu.VMEM((GS,), jnp.float32),
            pltpu.VMEM((L,), jnp.float32),
            pltpu.VMEM((L,), jnp.float32),
            pltpu.VMEM((G,), jnp.float32),
            pltpu.VMEM((G,), jnp.float32),
            pltpu.VMEM((CH,), jnp.float32),
            pltpu.SemaphoreType.DMA,
            pltpu.VMEM_SHARED((G,), jnp.float32),
            pltpu.VMEM_SHARED((G,), jnp.float32),
            pltpu.VMEM_SHARED((G,), jnp.float32),
            pltpu.VMEM_SHARED((G,), jnp.float32),
            pltpu.VMEM_SHARED((G,), jnp.float32),
            pltpu.VMEM_SHARED((L,), jnp.float32),
        ],
    )
    return f(rewards, gids_flat)


BLK = 4096
NB = N // BLK


def _tc_loss_body(obs_ref, act_ref, lpo_ref, adv_ref, w1_ref, b1_ref,
                  w2_ref, b2c_ref, lohi_ref, pg_ref, ent_ref):
    i = pl.program_id(0)
    x = obs_ref[...]
    h = jnp.tanh(jnp.dot(x, w1_ref[...], preferred_element_type=jnp.float32)
                 + b1_ref[...])
    lt = lax.dot_general(w2_ref[...], h, (((0,), (1,)), ((), ())),
                         preferred_element_type=jnp.float32) + b2c_ref[...]
    m = jnp.max(lt, axis=0, keepdims=True)
    e = jnp.exp(lt - m)
    se = jnp.sum(e, axis=0, keepdims=True)
    lse = jnp.log(se) + m
    act_row = act_ref[...].reshape(1, BLK)
    oh = lax.broadcasted_iota(jnp.int32, (ACT_DIM, BLK), 0) == act_row
    picked = jnp.sum(jnp.where(oh, lt, 0.0), axis=0, keepdims=True)
    ent_blk = jnp.sum(lse - jnp.sum(e * lt, axis=0, keepdims=True) / se)
    ratio = jnp.exp(picked - lse - lpo_ref[...].reshape(1, BLK))
    adv = adv_ref[...].reshape(1, BLK)
    lo = lohi_ref[0, 0]
    hi = lohi_ref[0, 1]
    s1 = ratio * adv
    s2 = jnp.clip(ratio, lo, hi) * adv
    pg_blk = jnp.sum(jnp.minimum(s1, s2))

    @pl.when(i == 0)
    def _():
        pg_ref[0, 0] = 0.0
        ent_ref[0, 0] = 0.0

    pg_ref[0, 0] += pg_blk
    ent_ref[0, 0] += ent_blk


def _tc_loss_call(obs, act1, lpo1, adv1, pW1, pb1r, pW2, pb2c, lohi):
    return pl.pallas_call(
        _tc_loss_body,
        grid=(NB,),
        in_specs=[
            pl.BlockSpec((BLK, OBS_DIM), lambda i: (i, 0)),
            pl.BlockSpec((BLK,), lambda i: (i,)),
            pl.BlockSpec((BLK,), lambda i: (i,)),
            pl.BlockSpec((BLK,), lambda i: (i,)),
            pl.BlockSpec((OBS_DIM, HID), lambda i: (0, 0)),
            pl.BlockSpec((1, HID), lambda i: (0, 0)),
            pl.BlockSpec((HID, ACT_DIM), lambda i: (0, 0)),
            pl.BlockSpec((ACT_DIM, 1), lambda i: (0, 0)),
            pl.BlockSpec((1, 2), lambda i: (0, 0), memory_space=pltpu.SMEM),
        ],
        out_specs=[
            pl.BlockSpec((1, 1), lambda i: (0, 0), memory_space=pltpu.SMEM),
            pl.BlockSpec((1, 1), lambda i: (0, 0), memory_space=pltpu.SMEM),
        ],
        out_shape=[
            jax.ShapeDtypeStruct((1, 1), jnp.float32),
            jax.ShapeDtypeStruct((1, 1), jnp.float32),
        ],
        compiler_params=pltpu.CompilerParams(
            dimension_semantics=("arbitrary",),
        ),
    )(obs, act1, lpo1, adv1, pW1, pb1r, pW2, pb2c, lohi)


def kernel(obs, act, rewards, group_ids, logp_old, pW1, pb1, pW2, pb2,
           vW1, vb1, vW2, vb2, clip_eps, ent_coef, beta_kl, ref_model):
    gids = group_ids.astype(jnp.int32)
    adv = _sc_adv_call(rewards, gids)
    ce = jnp.asarray(clip_eps, jnp.float32)
    lohi = jnp.stack([1.0 - ce, 1.0 + ce]).reshape(1, 2)
    pg_sum, ent_sum = _tc_loss_call(
        obs,
        act.astype(jnp.int32),
        logp_old,
        adv,
        pW1,
        pb1.reshape(1, HID),
        pW2,
        pb2.reshape(ACT_DIM, 1),
        lohi,
    )
    ec = jnp.asarray(ent_coef, jnp.float32)
    return -(pg_sum[0, 0] + ec * ent_sum[0, 0]) / N
